# initial kernel scaffold (unmeasured)
import jax
import jax.numpy as jnp
from jax import lax
from jax.experimental import pallas as pl
from jax.experimental.pallas import tpu as pltpu

N_DEV = 8
M_PER = 512
COMM_DTYPE = jnp.bfloat16


def kernel(x, w_mat, scale_x, scale_w):
    k, n = w_mat.shape
    m = x.shape[0]

    x_bf = x.astype(jnp.bfloat16)
    w_bf = w_mat.astype(jnp.bfloat16)
    scale = (scale_x[0] * scale_w[0]).reshape(1, 1).astype(jnp.float32)

    def body(x_ref, w_ref, scale_ref, out_ref,
             send_buf, recv_buf, send_sem, recv_sems, credit_sem):
        my = lax.axis_index("i")
        left = lax.rem(my - 1 + N_DEV, N_DEV)
        right = lax.rem(my + 1, N_DEV)

        barrier_sem = pltpu.get_barrier_semaphore()
        for nbr in (left, right):
            pl.semaphore_signal(
                barrier_sem, inc=1,
                device_id=(nbr,), device_id_type=pl.DeviceIdType.MESH,
            )
        pl.semaphore_wait(barrier_sem, 2)

        def partial(c):
            xs = x_ref[pl.ds(c * M_PER, M_PER), :]
            return lax.dot_general(
                xs, w_ref[:, :],
                dimension_numbers=(((1,), (0,)), ((), ())),
                preferred_element_type=jnp.float32,
            )

        for s in range(N_DEV - 1):
            c = lax.rem(my - 1 - s + 2 * N_DEV, N_DEV)
            p = partial(c)
            if s == 0:
                send_buf[:, :] = p.astype(COMM_DTYPE)
            else:
                acc = p + recv_buf[(s - 1) % 2].astype(jnp.float32)
                send_buf[:, :] = acc.astype(COMM_DTYPE)
                if s <= 5:
                    pl.semaphore_signal(
                        credit_sem, inc=1,
                        device_id=(left,),
                        device_id_type=pl.DeviceIdType.MESH,
                    )
            if s >= 2:
                pl.semaphore_wait(credit_sem, 1)
            rdma = pltpu.make_async_remote_copy(
                src_ref=send_buf,
                dst_ref=recv_buf.at[s % 2],
                send_sem=send_sem,
                recv_sem=recv_sems.at[s % 2],
                device_id=(right,),
                device_id_type=pl.DeviceIdType.MESH,
            )
            rdma.start()
            rdma.wait()

        p = partial(my)
        acc = p + recv_buf[(N_DEV - 2) % 2].astype(jnp.float32)
        out_ref[:, :] = jnp.maximum(acc * scale_ref[0, 0], 0.0)

    return pl.pallas_call(
        body,
        out_shape=jax.ShapeDtypeStruct((M_PER, n), jnp.float32),
        in_specs=[
            pl.BlockSpec(memory_space=pltpu.VMEM),
            pl.BlockSpec(memory_space=pltpu.VMEM),
            pl.BlockSpec(memory_space=pltpu.SMEM),
        ],
        out_specs=pl.BlockSpec(memory_space=pltpu.VMEM),
        scratch_shapes=[
            pltpu.VMEM((M_PER, n), COMM_DTYPE),
            pltpu.VMEM((2, M_PER, n), COMM_DTYPE),
            pltpu.SemaphoreType.DMA,
            pltpu.SemaphoreType.DMA((2,)),
            pltpu.SemaphoreType.REGULAR,
        ],
        compiler_params=pltpu.CompilerParams(collective_id=0),
    )(x_bf, w_bf, scale)


# baseline (device time: 722775 ns/iter reference)
import jax
import jax.numpy as jnp
from jax import lax
from jax.experimental import pallas as pl
from jax.experimental.pallas import tpu as pltpu

N_DEV = 8
M_PER = 512
COMM_DTYPE = jnp.bfloat16


def kernel(x, w_mat, scale_x, scale_w):
    k, n = w_mat.shape
    m = x.shape[0]

    x_bf = x.astype(jnp.bfloat16)
    w_bf = w_mat.astype(jnp.bfloat16)
    scale = (scale_x[0] * scale_w[0]).reshape(1, 1).astype(jnp.float32)

    def body(x_ref, w_ref, scale_ref, out_ref,
             send_buf, recv_buf, send_sem, recv_sems, credit_sem):
        my = lax.axis_index("i")
        left = lax.rem(my - 1 + N_DEV, N_DEV)
        right = lax.rem(my + 1, N_DEV)

        barrier_sem = pltpu.get_barrier_semaphore()
        for nbr in (left, right):
            pl.semaphore_signal(
                barrier_sem, inc=1,
                device_id=(nbr,), device_id_type=pl.DeviceIdType.MESH,
            )
        pl.semaphore_wait(barrier_sem, 2)

        def partial(c):
            xs = x_ref[pl.ds(c * M_PER, M_PER), :]
            return lax.dot_general(
                xs, w_ref[:, :],
                dimension_numbers=(((1,), (0,)), ((), ())),
                preferred_element_type=jnp.float32,
            )

        for s in range(N_DEV - 1):
            c = lax.rem(my - 1 - s + 2 * N_DEV, N_DEV)
            p = partial(c)
            if s == 0:
                send_buf[:, :] = p.astype(COMM_DTYPE)
            else:
                acc = p + recv_buf[(s - 1) % 2].astype(jnp.float32)
                send_buf[:, :] = acc.astype(COMM_DTYPE)
                if s <= 5:
                    pl.semaphore_signal(
                        credit_sem, inc=1,
                        device_id=(left,),
                        device_id_type=pl.DeviceIdType.MESH,
                    )
            if s >= 2:
                pl.semaphore_wait(credit_sem, 1)
            rdma = pltpu.make_async_remote_copy(
                src_ref=send_buf,
                dst_ref=recv_buf.at[s % 2],
                send_sem=send_sem,
                recv_sem=recv_sems.at[s % 2],
                device_id=(right,),
                device_id_type=pl.DeviceIdType.MESH,
            )
            rdma.start()
            rdma.wait()

        p = partial(my)
        acc = p + recv_buf[(N_DEV - 2) % 2].astype(jnp.float32)
        out_ref[:, :] = jnp.maximum(acc * scale_ref[0, 0], 0.0)

    return pl.pallas_call(
        body,
        out_shape=jax.ShapeDtypeStruct((M_PER, n), jnp.float32),
        in_specs=[
            pl.BlockSpec(memory_space=pltpu.VMEM),
            pl.BlockSpec(memory_space=pltpu.VMEM),
            pl.BlockSpec(memory_space=pltpu.SMEM),
        ],
        out_specs=pl.BlockSpec(memory_space=pltpu.VMEM),
        scratch_shapes=[
            pltpu.VMEM((M_PER, n), COMM_DTYPE),
            pltpu.VMEM((2, M_PER, n), COMM_DTYPE),
            pltpu.SemaphoreType.DMA,
            pltpu.SemaphoreType.DMA((2,)),
            pltpu.SemaphoreType.REGULAR,
        ],
        compiler_params=pltpu.CompilerParams(
            collective_id=0, vmem_limit_bytes=100 * 1024 * 1024
        ),
    )(x_bf, w_bf, scale)


# device time: 713434 ns/iter; 1.0131x vs baseline; 1.0131x over previous
import jax
import jax.numpy as jnp
from jax import lax
from jax.experimental import pallas as pl
from jax.experimental.pallas import tpu as pltpu

N_DEV = 8
M_PER = 512
NT = 4
COMM_DTYPE = jnp.bfloat16


def kernel(x, w_mat, scale_x, scale_w):
    k, n = w_mat.shape
    tn = n // NT

    x_bf = x.astype(jnp.bfloat16)
    w_bf = w_mat.astype(jnp.bfloat16)
    scale = (scale_x[0] * scale_w[0]).reshape(1, 1).astype(jnp.float32)

    def body(x_ref, w_ref, scale_ref, out_ref,
             buf, p_buf, send_sems, recv_sems, credit_sem, out_sem):
        my = lax.axis_index("i")
        left = lax.rem(my - 1 + N_DEV, N_DEV)
        right = lax.rem(my + 1, N_DEV)

        barrier_sem = pltpu.get_barrier_semaphore()
        for nbr in (left, right):
            pl.semaphore_signal(
                barrier_sem, inc=1,
                device_id=(nbr,), device_id_type=pl.DeviceIdType.MESH,
            )
        pl.semaphore_wait(barrier_sem, 2)

        def partial_into(c, dst, dst_slice=slice(None)):
            xs = x_ref[pl.ds(c * M_PER, M_PER), :]
            for t in range(NT):
                sl = pl.ds(t * tn, tn)
                dst[dst_slice, sl] = lax.dot_general(
                    xs, w_ref[:, sl],
                    dimension_numbers=(((1,), (0,)), ((), ())),
                    preferred_element_type=jnp.float32,
                ).astype(dst.dtype)

        partial_into(lax.rem(my - 1 + N_DEV, N_DEV), buf.at[1])

        rdmas = []
        for s in range(N_DEV - 1):
            if s >= 1:
                rdmas[s - 1].wait_send()
                pl.semaphore_signal(
                    credit_sem, inc=1,
                    device_id=(left,), device_id_type=pl.DeviceIdType.MESH,
                )
                pl.semaphore_wait(credit_sem, 1)
            rdma = pltpu.make_async_remote_copy(
                src_ref=buf.at[(s + 1) % 2],
                dst_ref=buf.at[s % 2],
                send_sem=send_sems.at[s % 2],
                recv_sem=recv_sems.at[s % 2],
                device_id=(right,),
                device_id_type=pl.DeviceIdType.MESH,
            )
            rdma.start()
            rdmas.append(rdma)
            partial_into(lax.rem(my - 2 - s + 2 * N_DEV, N_DEV), p_buf)
            rdma.wait_recv()
            if s < N_DEV - 2:
                for t in range(NT):
                    sl = pl.ds(t * tn, tn)
                    buf[s % 2, :, sl] = (
                        p_buf[:, sl] + buf[s % 2, :, sl].astype(jnp.float32)
                    ).astype(COMM_DTYPE)
            else:
                for t in range(NT):
                    sl = pl.ds(t * tn, tn)
                    p_buf[:, sl] = jnp.maximum(
                        (p_buf[:, sl]
                         + buf[s % 2, :, sl].astype(jnp.float32))
                        * scale_ref[0, 0],
                        0.0,
                    )
                out_copy = pltpu.make_async_copy(p_buf, out_ref, out_sem)
                out_copy.start()
                out_copy.wait()
        rdmas[N_DEV - 2].wait_send()

    return pl.pallas_call(
        body,
        out_shape=jax.ShapeDtypeStruct((M_PER, n), jnp.float32),
        in_specs=[
            pl.BlockSpec(memory_space=pltpu.VMEM),
            pl.BlockSpec(memory_space=pltpu.VMEM),
            pl.BlockSpec(memory_space=pltpu.SMEM),
        ],
        out_specs=pl.BlockSpec(memory_space=pltpu.MemorySpace.HBM),
        scratch_shapes=[
            pltpu.VMEM((2, M_PER, n), COMM_DTYPE),
            pltpu.VMEM((M_PER, n), jnp.float32),
            pltpu.SemaphoreType.DMA((2,)),
            pltpu.SemaphoreType.DMA((2,)),
            pltpu.SemaphoreType.REGULAR,
            pltpu.SemaphoreType.DMA,
        ],
        compiler_params=pltpu.CompilerParams(
            collective_id=0, vmem_limit_bytes=100 * 1024 * 1024
        ),
    )(x_bf, w_bf, scale)


# device time: 394426 ns/iter; 1.8325x vs baseline; 1.8088x over previous
import jax
import jax.numpy as jnp
from jax import lax
from jax.experimental import pallas as pl
from jax.experimental.pallas import tpu as pltpu

N_DEV = 8
M_PER = 512
TN = 2048
COMM_DTYPE = jnp.bfloat16


def kernel(x, w_mat, scale_x, scale_w):
    k, n = w_mat.shape
    half = n // 2

    x_bf = x.astype(jnp.bfloat16)
    w_bf = w_mat.astype(jnp.bfloat16)
    scale = (scale_x[0] * scale_w[0]).reshape(1, 1).astype(jnp.float32)

    def body(x_ref, w_ref, scale_ref, out_ref,
             buf_r, buf_l, p_buf,
             send_sems_r, recv_sems_r, send_sems_l, recv_sems_l,
             credit_r, credit_l, out_sem):
        my = lax.axis_index("i")
        left = lax.rem(my - 1 + N_DEV, N_DEV)
        right = lax.rem(my + 1, N_DEV)

        barrier_sem = pltpu.get_barrier_semaphore()
        for nbr in (left, right):
            pl.semaphore_signal(
                barrier_sem, inc=1,
                device_id=(nbr,), device_id_type=pl.DeviceIdType.MESH,
            )
        pl.semaphore_wait(barrier_sem, 2)

        def partial_half(c, off):
            xs = x_ref[pl.ds(c * M_PER, M_PER), :]
            for t in range(half // TN):
                sl = pl.ds(off + t * TN, TN)
                p_buf[:, sl] = lax.dot_general(
                    xs, w_ref[:, sl],
                    dimension_numbers=(((1,), (0,)), ((), ())),
                    preferred_element_type=jnp.float32,
                )

        def accumulate(dst, off, epilogue=False):
            for t in range(half // TN):
                dsl = pl.ds(t * TN, TN)
                psl = pl.ds(off + t * TN, TN)
                acc = p_buf[:, psl] + dst[:, dsl].astype(jnp.float32)
                if epilogue:
                    p_buf[:, psl] = jnp.maximum(acc * scale_ref[0, 0], 0.0)
                else:
                    dst[:, dsl] = acc.astype(COMM_DTYPE)

        c0r = lax.rem(my - 1 + N_DEV, N_DEV)
        c0l = lax.rem(my + 1, N_DEV)
        xs = x_ref[pl.ds(c0r * M_PER, M_PER), :]
        for t in range(half // TN):
            sl = pl.ds(t * TN, TN)
            buf_r[1, :, sl] = lax.dot_general(
                xs, w_ref[:, sl],
                dimension_numbers=(((1,), (0,)), ((), ())),
                preferred_element_type=jnp.float32,
            ).astype(COMM_DTYPE)
        xs = x_ref[pl.ds(c0l * M_PER, M_PER), :]
        for t in range(half // TN):
            sl = pl.ds(t * TN, TN)
            buf_l[1, :, sl] = lax.dot_general(
                xs, w_ref[:, pl.ds(half + t * TN, TN)],
                dimension_numbers=(((1,), (0,)), ((), ())),
                preferred_element_type=jnp.float32,
            ).astype(COMM_DTYPE)

        rdmas_r = []
        rdmas_l = []
        for s in range(N_DEV - 1):
            if s >= 1:
                rdmas_r[s - 1].wait_send()
                pl.semaphore_signal(
                    credit_r, inc=1,
                    device_id=(left,), device_id_type=pl.DeviceIdType.MESH,
                )
                rdmas_l[s - 1].wait_send()
                pl.semaphore_signal(
                    credit_l, inc=1,
                    device_id=(right,), device_id_type=pl.DeviceIdType.MESH,
                )
                pl.semaphore_wait(credit_r, 1)
                pl.semaphore_wait(credit_l, 1)
            rdma_r = pltpu.make_async_remote_copy(
                src_ref=buf_r.at[(s + 1) % 2],
                dst_ref=buf_r.at[s % 2],
                send_sem=send_sems_r.at[s % 2],
                recv_sem=recv_sems_r.at[s % 2],
                device_id=(right,),
                device_id_type=pl.DeviceIdType.MESH,
            )
            rdma_l = pltpu.make_async_remote_copy(
                src_ref=buf_l.at[(s + 1) % 2],
                dst_ref=buf_l.at[s % 2],
                send_sem=send_sems_l.at[s % 2],
                recv_sem=recv_sems_l.at[s % 2],
                device_id=(left,),
                device_id_type=pl.DeviceIdType.MESH,
            )
            rdma_r.start()
            rdma_l.start()
            rdmas_r.append(rdma_r)
            rdmas_l.append(rdma_l)
            partial_half(lax.rem(my - 2 - s + 2 * N_DEV, N_DEV), 0)
            partial_half(lax.rem(my + 2 + s, N_DEV), half)
            rdma_r.wait_recv()
            accumulate(buf_r.at[s % 2], 0, epilogue=(s == N_DEV - 2))
            rdma_l.wait_recv()
            accumulate(buf_l.at[s % 2], half, epilogue=(s == N_DEV - 2))
        out_copy = pltpu.make_async_copy(p_buf, out_ref, out_sem)
        out_copy.start()
        out_copy.wait()
        rdmas_r[N_DEV - 2].wait_send()
        rdmas_l[N_DEV - 2].wait_send()

    return pl.pallas_call(
        body,
        out_shape=jax.ShapeDtypeStruct((M_PER, n), jnp.float32),
        in_specs=[
            pl.BlockSpec(memory_space=pltpu.VMEM),
            pl.BlockSpec(memory_space=pltpu.VMEM),
            pl.BlockSpec(memory_space=pltpu.SMEM),
        ],
        out_specs=pl.BlockSpec(memory_space=pltpu.MemorySpace.HBM),
        scratch_shapes=[
            pltpu.VMEM((2, M_PER, half), COMM_DTYPE),
            pltpu.VMEM((2, M_PER, half), COMM_DTYPE),
            pltpu.VMEM((M_PER, n), jnp.float32),
            pltpu.SemaphoreType.DMA((2,)),
            pltpu.SemaphoreType.DMA((2,)),
            pltpu.SemaphoreType.DMA((2,)),
            pltpu.SemaphoreType.DMA((2,)),
            pltpu.SemaphoreType.REGULAR,
            pltpu.SemaphoreType.REGULAR,
            pltpu.SemaphoreType.DMA,
        ],
        compiler_params=pltpu.CompilerParams(
            collective_id=0, vmem_limit_bytes=100 * 1024 * 1024
        ),
    )(x_bf, w_bf, scale)


# device time: 385609 ns/iter; 1.8744x vs baseline; 1.0229x over previous
import jax
import jax.numpy as jnp
from jax import lax
from jax.experimental import pallas as pl
from jax.experimental.pallas import tpu as pltpu

N_DEV = 8
M_PER = 512
TN = 2048
COMM_DTYPE = jnp.bfloat16


def kernel(x, w_mat, scale_x, scale_w):
    k, n = w_mat.shape
    half = n // 2

    x_bf = x.astype(jnp.bfloat16)
    w_bf = w_mat.astype(jnp.bfloat16)
    scale = (scale_x[0] * scale_w[0]).reshape(1, 1).astype(jnp.float32)

    def body(x_ref, w_ref, scale_ref, out_ref,
             buf_r, buf_l, p_buf,
             send_sems_r, recv_sems_r, send_sems_l, recv_sems_l,
             credit_r, credit_l, out_sems):
        my = lax.axis_index("i")
        left = lax.rem(my - 1 + N_DEV, N_DEV)
        right = lax.rem(my + 1, N_DEV)

        barrier_sem = pltpu.get_barrier_semaphore()
        for nbr in (left, right):
            pl.semaphore_signal(
                barrier_sem, inc=1,
                device_id=(nbr,), device_id_type=pl.DeviceIdType.MESH,
            )
        pl.semaphore_wait(barrier_sem, 2)

        def partial_half(c, off, dst, dst_off):
            xs = x_ref[pl.ds(c * M_PER, M_PER), :]
            for t in range(half // TN):
                p = lax.dot_general(
                    xs, w_ref[:, pl.ds(off + t * TN, TN)],
                    dimension_numbers=(((1,), (0,)), ((), ())),
                    preferred_element_type=jnp.float32,
                )
                dst[:, pl.ds(dst_off + t * TN, TN)] = p.astype(dst.dtype)

        def accumulate(dst, off, epilogue=False):
            for t in range(half // TN):
                dsl = pl.ds(t * TN, TN)
                psl = pl.ds(off + t * TN, TN)
                acc = p_buf[:, psl] + dst[:, dsl].astype(jnp.float32)
                if epilogue:
                    p_buf[:, psl] = jnp.maximum(acc * scale_ref[0, 0], 0.0)
                else:
                    dst[:, dsl] = acc.astype(COMM_DTYPE)

        partial_half(lax.rem(my - 1 + N_DEV, N_DEV), 0, buf_r.at[2], 0)
        partial_half(lax.rem(my + 1, N_DEV), half, buf_l.at[2], 0)

        rdmas_r = []
        rdmas_l = []
        for s in range(N_DEV - 1):
            src = 2 if s == 0 else (s - 1) % 3
            if s >= 2:
                pl.semaphore_wait(credit_r, 1)
                pl.semaphore_wait(credit_l, 1)
            rdma_r = pltpu.make_async_remote_copy(
                src_ref=buf_r.at[src],
                dst_ref=buf_r.at[s % 3],
                send_sem=send_sems_r.at[s % 3],
                recv_sem=recv_sems_r.at[s % 3],
                device_id=(right,),
                device_id_type=pl.DeviceIdType.MESH,
            )
            rdma_l = pltpu.make_async_remote_copy(
                src_ref=buf_l.at[src],
                dst_ref=buf_l.at[s % 3],
                send_sem=send_sems_l.at[s % 3],
                recv_sem=recv_sems_l.at[s % 3],
                device_id=(left,),
                device_id_type=pl.DeviceIdType.MESH,
            )
            rdma_r.start()
            rdma_l.start()
            rdmas_r.append(rdma_r)
            rdmas_l.append(rdma_l)
            if 1 <= s <= 5:
                rdmas_r[s - 1].wait_send()
                pl.semaphore_signal(
                    credit_r, inc=1,
                    device_id=(left,), device_id_type=pl.DeviceIdType.MESH,
                )
                rdmas_l[s - 1].wait_send()
                pl.semaphore_signal(
                    credit_l, inc=1,
                    device_id=(right,), device_id_type=pl.DeviceIdType.MESH,
                )
            partial_half(lax.rem(my - 2 - s + 2 * N_DEV, N_DEV), 0, p_buf, 0)
            partial_half(lax.rem(my + 2 + s, N_DEV), half, p_buf, half)
            last = s == N_DEV - 2
            rdma_r.wait_recv()
            accumulate(buf_r.at[s % 3], 0, epilogue=last)
            rdma_l.wait_recv()
            accumulate(buf_l.at[s % 3], half, epilogue=last)
        out_copy = pltpu.make_async_copy(p_buf, out_ref, out_sems.at[0])
        out_copy.start()
        out_copy.wait()
        for d in (rdmas_r, rdmas_l):
            d[N_DEV - 3].wait_send()
            d[N_DEV - 2].wait_send()

    return pl.pallas_call(
        body,
        out_shape=jax.ShapeDtypeStruct((M_PER, n), jnp.float32),
        in_specs=[
            pl.BlockSpec(memory_space=pltpu.VMEM),
            pl.BlockSpec(memory_space=pltpu.VMEM),
            pl.BlockSpec(memory_space=pltpu.SMEM),
        ],
        out_specs=pl.BlockSpec(memory_space=pltpu.MemorySpace.HBM),
        scratch_shapes=[
            pltpu.VMEM((3, M_PER, half), COMM_DTYPE),
            pltpu.VMEM((3, M_PER, half), COMM_DTYPE),
            pltpu.VMEM((M_PER, n), jnp.float32),
            pltpu.SemaphoreType.DMA((3,)),
            pltpu.SemaphoreType.DMA((3,)),
            pltpu.SemaphoreType.DMA((3,)),
            pltpu.SemaphoreType.DMA((3,)),
            pltpu.SemaphoreType.REGULAR,
            pltpu.SemaphoreType.REGULAR,
            pltpu.SemaphoreType.DMA((2,)),
        ],
        compiler_params=pltpu.CompilerParams(
            collective_id=0, vmem_limit_bytes=100 * 1024 * 1024
        ),
    )(x_bf, w_bf, scale)


# device time: 378009 ns/iter; 1.9121x vs baseline; 1.0201x over previous
import jax
import jax.numpy as jnp
from jax import lax
from jax.experimental import pallas as pl
from jax.experimental.pallas import tpu as pltpu

N_DEV = 8
M_PER = 512
TN = 2048
COMM_DTYPE = jnp.bfloat16


def kernel(x, w_mat, scale_x, scale_w):
    k, n = w_mat.shape
    half = n // 2
    ntiles = half // TN

    x_bf = x.astype(jnp.bfloat16)
    w_bf = w_mat.astype(jnp.bfloat16)
    scale = (scale_x[0] * scale_w[0]).reshape(1, 1).astype(jnp.float32)

    def body(x_ref, w_ref, scale_ref, out_ref,
             buf_r, buf_l, p_buf,
             send_sems_r, recv_sems_r, send_sems_l, recv_sems_l,
             credit_r, credit_l, out_sem):
        my = lax.axis_index("i")
        left = lax.rem(my - 1 + N_DEV, N_DEV)
        right = lax.rem(my + 1, N_DEV)

        barrier_sem = pltpu.get_barrier_semaphore()
        for nbr in (left, right):
            pl.semaphore_signal(
                barrier_sem, inc=1,
                device_id=(nbr,), device_id_type=pl.DeviceIdType.MESH,
            )
        pl.semaphore_wait(barrier_sem, 2)

        def gemm_tile(c, off, t):
            xs = x_ref[pl.ds(c * M_PER, M_PER), :]
            return lax.dot_general(
                xs, w_ref[:, pl.ds(off + t * TN, TN)],
                dimension_numbers=(((1,), (0,)), ((), ())),
                preferred_element_type=jnp.float32,
            )

        def make_tile_rdma(bufs, s, src, t, sends, recvs, nbr):
            return pltpu.make_async_remote_copy(
                src_ref=bufs.at[src, t],
                dst_ref=bufs.at[s % 3, t],
                send_sem=sends.at[s % 3, t],
                recv_sem=recvs.at[s % 3, t],
                device_id=(nbr,),
                device_id_type=pl.DeviceIdType.MESH,
            )

        def acc_tile(bufs, s, off, t, epilogue):
            psl = pl.ds(off + t * TN, TN)
            acc = p_buf[:, psl] + bufs[s % 3, t].astype(jnp.float32)
            if epilogue:
                p_buf[:, psl] = jnp.maximum(acc * scale_ref[0, 0], 0.0)
            else:
                bufs[s % 3, t] = acc.astype(COMM_DTYPE)

        c0r = lax.rem(my - 1 + N_DEV, N_DEV)
        c0l = lax.rem(my + 1, N_DEV)
        rdmas_r = [[None] * ntiles for _ in range(N_DEV - 1)]
        rdmas_l = [[None] * ntiles for _ in range(N_DEV - 1)]
        for t in range(ntiles):
            buf_r[2, t] = gemm_tile(c0r, 0, t).astype(COMM_DTYPE)
            rdmas_r[0][t] = make_tile_rdma(
                buf_r, 0, 2, t, send_sems_r, recv_sems_r, right)
            rdmas_r[0][t].start()
            buf_l[2, t] = gemm_tile(c0l, half, t).astype(COMM_DTYPE)
            rdmas_l[0][t] = make_tile_rdma(
                buf_l, 0, 2, t, send_sems_l, recv_sems_l, left)
            rdmas_l[0][t].start()

        for s in range(N_DEV - 1):
            if s >= 1:
                src = (s - 1) % 3
                if s >= 2:
                    pl.semaphore_wait(credit_r, 1)
                    pl.semaphore_wait(credit_l, 1)
                for t in range(ntiles):
                    rdmas_r[s][t] = make_tile_rdma(
                        buf_r, s, src, t, send_sems_r, recv_sems_r, right)
                    rdmas_r[s][t].start()
                    rdmas_l[s][t] = make_tile_rdma(
                        buf_l, s, src, t, send_sems_l, recv_sems_l, left)
                    rdmas_l[s][t].start()
            if 1 <= s <= 5:
                for t in range(ntiles):
                    rdmas_r[s - 1][t].wait_send()
                    rdmas_l[s - 1][t].wait_send()
                pl.semaphore_signal(
                    credit_r, inc=1,
                    device_id=(left,), device_id_type=pl.DeviceIdType.MESH,
                )
                pl.semaphore_signal(
                    credit_l, inc=1,
                    device_id=(right,), device_id_type=pl.DeviceIdType.MESH,
                )
            cr = lax.rem(my - 2 - s + 2 * N_DEV, N_DEV)
            cl = lax.rem(my + 2 + s, N_DEV)
            for t in range(ntiles):
                p_buf[:, pl.ds(t * TN, TN)] = gemm_tile(cr, 0, t)
                p_buf[:, pl.ds(half + t * TN, TN)] = gemm_tile(cl, half, t)
            last = s == N_DEV - 2
            for t in range(ntiles):
                rdmas_r[s][t].wait_recv()
                acc_tile(buf_r, s, 0, t, last)
                rdmas_l[s][t].wait_recv()
                acc_tile(buf_l, s, half, t, last)
        out_copy = pltpu.make_async_copy(p_buf, out_ref, out_sem)
        out_copy.start()
        out_copy.wait()
        for d in (rdmas_r, rdmas_l):
            for t in range(ntiles):
                d[N_DEV - 3][t].wait_send()
                d[N_DEV - 2][t].wait_send()

    return pl.pallas_call(
        body,
        out_shape=jax.ShapeDtypeStruct((M_PER, n), jnp.float32),
        in_specs=[
            pl.BlockSpec(memory_space=pltpu.VMEM),
            pl.BlockSpec(memory_space=pltpu.VMEM),
            pl.BlockSpec(memory_space=pltpu.SMEM),
        ],
        out_specs=pl.BlockSpec(memory_space=pltpu.MemorySpace.HBM),
        scratch_shapes=[
            pltpu.VMEM((3, 2, M_PER, TN), COMM_DTYPE),
            pltpu.VMEM((3, 2, M_PER, TN), COMM_DTYPE),
            pltpu.VMEM((M_PER, n), jnp.float32),
            pltpu.SemaphoreType.DMA((3, 2)),
            pltpu.SemaphoreType.DMA((3, 2)),
            pltpu.SemaphoreType.DMA((3, 2)),
            pltpu.SemaphoreType.DMA((3, 2)),
            pltpu.SemaphoreType.REGULAR,
            pltpu.SemaphoreType.REGULAR,
            pltpu.SemaphoreType.DMA,
        ],
        compiler_params=pltpu.CompilerParams(
            collective_id=0, vmem_limit_bytes=100 * 1024 * 1024
        ),
    )(x_bf, w_bf, scale)


# device time: 374892 ns/iter; 1.9280x vs baseline; 1.0083x over previous
import jax
import jax.numpy as jnp
from jax import lax
from jax.experimental import pallas as pl
from jax.experimental.pallas import tpu as pltpu

N_DEV = 8
M_PER = 512
TN = 1024
COMM_DTYPE = jnp.bfloat16


def kernel(x, w_mat, scale_x, scale_w):
    k, n = w_mat.shape
    half = n // 2
    ntiles = half // TN

    x_bf = x.astype(jnp.bfloat16)
    w_bf = w_mat.astype(jnp.bfloat16)
    scale = (scale_x[0] * scale_w[0]).reshape(1, 1).astype(jnp.float32)

    def body(x_ref, w_ref, scale_ref, out_ref,
             buf_r, buf_l, p_buf,
             send_sems_r, recv_sems_r, send_sems_l, recv_sems_l,
             credit_r, credit_l, out_sem):
        my = lax.axis_index("i")
        left = lax.rem(my - 1 + N_DEV, N_DEV)
        right = lax.rem(my + 1, N_DEV)

        barrier_sem = pltpu.get_barrier_semaphore()
        for nbr in (left, right):
            pl.semaphore_signal(
                barrier_sem, inc=1,
                device_id=(nbr,), device_id_type=pl.DeviceIdType.MESH,
            )
        pl.semaphore_wait(barrier_sem, 2)

        def gemm_tile(c, off, t):
            xs = x_ref[pl.ds(c * M_PER, M_PER), :]
            return lax.dot_general(
                xs, w_ref[:, pl.ds(off + t * TN, TN)],
                dimension_numbers=(((1,), (0,)), ((), ())),
                preferred_element_type=jnp.float32,
            )

        def make_tile_rdma(bufs, s, src, t, sends, recvs, nbr):
            return pltpu.make_async_remote_copy(
                src_ref=bufs.at[src, t],
                dst_ref=bufs.at[s % 3, t],
                send_sem=sends.at[s % 3, t],
                recv_sem=recvs.at[s % 3, t],
                device_id=(nbr,),
                device_id_type=pl.DeviceIdType.MESH,
            )

        def acc_tile(bufs, s, off, t, epilogue):
            psl = pl.ds(off + t * TN, TN)
            acc = p_buf[:, psl] + bufs[s % 3, t].astype(jnp.float32)
            if epilogue:
                p_buf[:, psl] = jnp.maximum(acc * scale_ref[0, 0], 0.0)
            else:
                bufs[s % 3, t] = acc.astype(COMM_DTYPE)

        c0r = lax.rem(my - 1 + N_DEV, N_DEV)
        c0l = lax.rem(my + 1, N_DEV)
        rdmas_r = [[None] * ntiles for _ in range(N_DEV - 1)]
        rdmas_l = [[None] * ntiles for _ in range(N_DEV - 1)]
        for t in range(ntiles):
            buf_r[2, t] = gemm_tile(c0r, 0, t).astype(COMM_DTYPE)
            rdmas_r[0][t] = make_tile_rdma(
                buf_r, 0, 2, t, send_sems_r, recv_sems_r, right)
            rdmas_r[0][t].start()
            buf_l[2, t] = gemm_tile(c0l, half, t).astype(COMM_DTYPE)
            rdmas_l[0][t] = make_tile_rdma(
                buf_l, 0, 2, t, send_sems_l, recv_sems_l, left)
            rdmas_l[0][t].start()

        for s in range(N_DEV - 1):
            if s >= 1:
                src = (s - 1) % 3
                if s >= 2:
                    pl.semaphore_wait(credit_r, 1)
                    pl.semaphore_wait(credit_l, 1)
                for t in range(ntiles):
                    rdmas_r[s][t] = make_tile_rdma(
                        buf_r, s, src, t, send_sems_r, recv_sems_r, right)
                    rdmas_r[s][t].start()
                    rdmas_l[s][t] = make_tile_rdma(
                        buf_l, s, src, t, send_sems_l, recv_sems_l, left)
                    rdmas_l[s][t].start()
            if 1 <= s <= 5:
                for t in range(ntiles):
                    rdmas_r[s - 1][t].wait_send()
                    rdmas_l[s - 1][t].wait_send()
                pl.semaphore_signal(
                    credit_r, inc=1,
                    device_id=(left,), device_id_type=pl.DeviceIdType.MESH,
                )
                pl.semaphore_signal(
                    credit_l, inc=1,
                    device_id=(right,), device_id_type=pl.DeviceIdType.MESH,
                )
            cr = lax.rem(my - 2 - s + 2 * N_DEV, N_DEV)
            cl = lax.rem(my + 2 + s, N_DEV)
            for t in range(ntiles):
                p_buf[:, pl.ds(t * TN, TN)] = gemm_tile(cr, 0, t)
                p_buf[:, pl.ds(half + t * TN, TN)] = gemm_tile(cl, half, t)
            last = s == N_DEV - 2
            for t in range(ntiles):
                rdmas_r[s][t].wait_recv()
                acc_tile(buf_r, s, 0, t, last)
                rdmas_l[s][t].wait_recv()
                acc_tile(buf_l, s, half, t, last)
        out_copy = pltpu.make_async_copy(p_buf, out_ref, out_sem)
        out_copy.start()
        out_copy.wait()
        for d in (rdmas_r, rdmas_l):
            for t in range(ntiles):
                d[N_DEV - 3][t].wait_send()
                d[N_DEV - 2][t].wait_send()

    return pl.pallas_call(
        body,
        out_shape=jax.ShapeDtypeStruct((M_PER, n), jnp.float32),
        in_specs=[
            pl.BlockSpec(memory_space=pltpu.VMEM),
            pl.BlockSpec(memory_space=pltpu.VMEM),
            pl.BlockSpec(memory_space=pltpu.SMEM),
        ],
        out_specs=pl.BlockSpec(memory_space=pltpu.MemorySpace.HBM),
        scratch_shapes=[
            pltpu.VMEM((3, 4, M_PER, TN), COMM_DTYPE),
            pltpu.VMEM((3, 4, M_PER, TN), COMM_DTYPE),
            pltpu.VMEM((M_PER, n), jnp.float32),
            pltpu.SemaphoreType.DMA((3, 4)),
            pltpu.SemaphoreType.DMA((3, 4)),
            pltpu.SemaphoreType.DMA((3, 4)),
            pltpu.SemaphoreType.DMA((3, 4)),
            pltpu.SemaphoreType.REGULAR,
            pltpu.SemaphoreType.REGULAR,
            pltpu.SemaphoreType.DMA,
        ],
        compiler_params=pltpu.CompilerParams(
            collective_id=0, vmem_limit_bytes=100 * 1024 * 1024
        ),
    )(x_bf, w_bf, scale)


# device time: 360525 ns/iter; 2.0048x vs baseline; 1.0399x over previous
import jax
import jax.numpy as jnp
from jax import lax
from jax.experimental import pallas as pl
from jax.experimental.pallas import tpu as pltpu

N_DEV = 8
M_PER = 512
TN = 1024
COMM_DTYPE = jnp.bfloat16


def kernel(x, w_mat, scale_x, scale_w):
    k, n = w_mat.shape
    half = n // 2
    ntiles = half // TN

    x_bf = x.astype(jnp.bfloat16)
    w_bf = w_mat.astype(jnp.bfloat16)
    scale = (scale_x[0] * scale_w[0]).reshape(1, 1).astype(jnp.float32)

    def body(x_ref, w_ref, scale_ref, out_ref,
             buf_r, buf_l, p_buf,
             send_sems_r, recv_sems_r, send_sems_l, recv_sems_l,
             credit_r, credit_l, out_sem):
        my = lax.axis_index("i")
        left = lax.rem(my - 1 + N_DEV, N_DEV)
        right = lax.rem(my + 1, N_DEV)

        barrier_sem = pltpu.get_barrier_semaphore()
        for nbr in (left, right):
            pl.semaphore_signal(
                barrier_sem, inc=1,
                device_id=(nbr,), device_id_type=pl.DeviceIdType.MESH,
            )
        pl.semaphore_wait(barrier_sem, 2)

        def gemm_tile(c, off, t):
            xs = x_ref[pl.ds(c * M_PER, M_PER), :]
            return lax.dot_general(
                xs, w_ref[:, pl.ds(off + t * TN, TN)],
                dimension_numbers=(((1,), (0,)), ((), ())),
                preferred_element_type=jnp.float32,
            )

        def make_tile_rdma(bufs, s, src, t, sends, recvs, nbr):
            return pltpu.make_async_remote_copy(
                src_ref=bufs.at[src, t],
                dst_ref=bufs.at[s % 3, t],
                send_sem=sends.at[s % 3, t],
                recv_sem=recvs.at[s % 3, t],
                device_id=(nbr,),
                device_id_type=pl.DeviceIdType.MESH,
            )

        def acc_tile(bufs, s, off, t, epilogue):
            psl = pl.ds(off + t * TN, TN)
            acc = p_buf[:, psl] + bufs[s % 3, t].astype(jnp.float32)
            if epilogue:
                p_buf[:, psl] = jnp.maximum(acc * scale_ref[0, 0], 0.0)
            else:
                bufs[s % 3, t] = acc.astype(COMM_DTYPE)

        c0r = lax.rem(my - 1 + N_DEV, N_DEV)
        c0l = lax.rem(my + 1, N_DEV)
        rdmas_r = [[None] * ntiles for _ in range(N_DEV - 1)]
        rdmas_l = [[None] * ntiles for _ in range(N_DEV - 1)]
        for t in range(ntiles):
            buf_r[2, t] = gemm_tile(c0r, 0, t).astype(COMM_DTYPE)
            rdmas_r[0][t] = make_tile_rdma(
                buf_r, 0, 2, t, send_sems_r, recv_sems_r, right)
            rdmas_r[0][t].start()
            buf_l[2, t] = gemm_tile(c0l, half, t).astype(COMM_DTYPE)
            rdmas_l[0][t] = make_tile_rdma(
                buf_l, 0, 2, t, send_sems_l, recv_sems_l, left)
            rdmas_l[0][t].start()

        for s in range(N_DEV - 1):
            if 1 <= s <= 5:
                for t in range(ntiles):
                    rdmas_r[s - 1][t].wait_send()
                    rdmas_l[s - 1][t].wait_send()
                pl.semaphore_signal(
                    credit_r, inc=1,
                    device_id=(left,), device_id_type=pl.DeviceIdType.MESH,
                )
                pl.semaphore_signal(
                    credit_l, inc=1,
                    device_id=(right,), device_id_type=pl.DeviceIdType.MESH,
                )
            cr = lax.rem(my - 2 - s + 2 * N_DEV, N_DEV)
            cl = lax.rem(my + 2 + s, N_DEV)
            for t in range(ntiles):
                p_buf[:, pl.ds(t * TN, TN)] = gemm_tile(cr, 0, t)
                p_buf[:, pl.ds(half + t * TN, TN)] = gemm_tile(cl, half, t)
            if 1 <= s <= 5:
                pl.semaphore_wait(credit_r, 1)
                pl.semaphore_wait(credit_l, 1)
            last = s == N_DEV - 2
            for t in range(ntiles):
                rdmas_r[s][t].wait_recv()
                acc_tile(buf_r, s, 0, t, last)
                if not last:
                    rdmas_r[s + 1][t] = make_tile_rdma(
                        buf_r, s + 1, s % 3, t,
                        send_sems_r, recv_sems_r, right)
                    rdmas_r[s + 1][t].start()
                rdmas_l[s][t].wait_recv()
                acc_tile(buf_l, s, half, t, last)
                if not last:
                    rdmas_l[s + 1][t] = make_tile_rdma(
                        buf_l, s + 1, s % 3, t,
                        send_sems_l, recv_sems_l, left)
                    rdmas_l[s + 1][t].start()
        out_copy = pltpu.make_async_copy(p_buf, out_ref, out_sem)
        out_copy.start()
        out_copy.wait()
        for d in (rdmas_r, rdmas_l):
            for t in range(ntiles):
                d[N_DEV - 3][t].wait_send()
                d[N_DEV - 2][t].wait_send()

    return pl.pallas_call(
        body,
        out_shape=jax.ShapeDtypeStruct((M_PER, n), jnp.float32),
        in_specs=[
            pl.BlockSpec(memory_space=pltpu.VMEM),
            pl.BlockSpec(memory_space=pltpu.VMEM),
            pl.BlockSpec(memory_space=pltpu.SMEM),
        ],
        out_specs=pl.BlockSpec(memory_space=pltpu.MemorySpace.HBM),
        scratch_shapes=[
            pltpu.VMEM((3, 4, M_PER, TN), COMM_DTYPE),
            pltpu.VMEM((3, 4, M_PER, TN), COMM_DTYPE),
            pltpu.VMEM((M_PER, n), jnp.float32),
            pltpu.SemaphoreType.DMA((3, 4)),
            pltpu.SemaphoreType.DMA((3, 4)),
            pltpu.SemaphoreType.DMA((3, 4)),
            pltpu.SemaphoreType.DMA((3, 4)),
            pltpu.SemaphoreType.REGULAR,
            pltpu.SemaphoreType.REGULAR,
            pltpu.SemaphoreType.DMA,
        ],
        compiler_params=pltpu.CompilerParams(
            collective_id=0, vmem_limit_bytes=100 * 1024 * 1024
        ),
    )(x_bf, w_bf, scale)
